# initial kernel scaffold (unmeasured)
import jax
import jax.numpy as jnp
from jax import lax
from jax.experimental import pallas as pl
from jax.experimental.pallas import tpu as pltpu

N_DEV = 4
S_PER = 2048
S_GLOB = N_DEV * S_PER
HQ = 8
DH = 128
DM = HQ * DH
QBLK = 256
WIN = QBLK + 256
NGLOB = 32
SCALE = 0.08838834764831843
NEG = -1e9


def kernel(x, Wq, K_ext, V_ext, Wo):
    k2 = K_ext.reshape(S_PER, DM)
    v2 = V_ext.reshape(S_PER, DM)

    def body(x_ref, wq_ref, k_ref, v_ref, wo_ref, out_ref,
             kfull, vfull, qbuf, ctx, send_sems, recv_sems):
        my = lax.axis_index("i")
        left = lax.rem(my + N_DEV - 1, N_DEV)
        right = lax.rem(my + 1, N_DEV)

        barrier = pltpu.get_barrier_semaphore()
        for nbr in (left, right):
            pl.semaphore_signal(barrier, inc=1, device_id=(nbr,),
                                device_id_type=pl.DeviceIdType.MESH)
        pl.semaphore_wait(barrier, 2)

        kfull[pl.ds(my * S_PER, S_PER), :] = k_ref[...].astype(jnp.bfloat16)
        vfull[pl.ds(my * S_PER, S_PER), :] = v_ref[...].astype(jnp.bfloat16)

        qbuf[...] = jnp.dot(
            x_ref[0].astype(jnp.bfloat16), wq_ref[...].astype(jnp.bfloat16),
            preferred_element_type=jnp.float32,
        ).astype(jnp.bfloat16)

        for h in range(N_DEV - 1):
            origin = lax.rem(my - h + 2 * N_DEV, N_DEV)
            row0 = origin * S_PER
            for t, buf in enumerate((kfull, vfull)):
                rdma = pltpu.make_async_remote_copy(
                    src_ref=buf.at[pl.ds(row0, S_PER), :],
                    dst_ref=buf.at[pl.ds(row0, S_PER), :],
                    send_sem=send_sems.at[t, h],
                    recv_sem=recv_sems.at[t, h],
                    device_id=(right,),
                    device_id_type=pl.DeviceIdType.MESH,
                )
                rdma.start()
                rdma.wait()

        for hh in range(HQ):
            c0 = hh * DH
            kg = kfull[0:NGLOB, c0:c0 + DH]
            vg = vfull[0:NGLOB, c0:c0 + DH]

            def qblock(qb, _, c0=c0, kg=kg, vg=vg):
                off = my * S_PER + qb * QBLK
                start = jnp.clip(off - 128, 0, S_GLOB - WIN)
                qh = qbuf[pl.ds(qb * QBLK, QBLK), c0:c0 + DH]
                kw = kfull[pl.ds(start, WIN), c0:c0 + DH]
                vw = vfull[pl.ds(start, WIN), c0:c0 + DH]
                sb = lax.dot_general(
                    qh, kw, (((1,), (1,)), ((), ())),
                    preferred_element_type=jnp.float32) * SCALE
                sg = lax.dot_general(
                    qh, kg, (((1,), (1,)), ((), ())),
                    preferred_element_type=jnp.float32) * SCALE
                rows = off + lax.broadcasted_iota(jnp.int32, (QBLK, WIN), 0)
                cols = start + lax.broadcasted_iota(jnp.int32, (QBLK, WIN), 1)
                ok_b = (jnp.abs(rows - cols) <= 128) | (cols < NGLOB)
                sb = jnp.where(ok_b, sb, NEG)
                gcols = lax.broadcasted_iota(jnp.int32, (QBLK, NGLOB), 1)
                sg = jnp.where(gcols < start, sg, NEG)
                m = jnp.maximum(jnp.max(sb, axis=1), jnp.max(sg, axis=1))
                wb = jnp.exp(sb - m[:, None])
                wg = jnp.exp(sg - m[:, None])
                denom = jnp.sum(wb, axis=1) + jnp.sum(wg, axis=1)
                acc = lax.dot_general(
                    wb.astype(jnp.bfloat16), vw, (((1,), (0,)), ((), ())),
                    preferred_element_type=jnp.float32)
                acc += lax.dot_general(
                    wg.astype(jnp.bfloat16), vg, (((1,), (0,)), ((), ())),
                    preferred_element_type=jnp.float32)
                ctx[pl.ds(qb * QBLK, QBLK), c0:c0 + DH] = acc / denom[:, None]
                return 0

            lax.fori_loop(0, S_PER // QBLK, qblock, 0)

        @pl.when(my == 0)
        def _():
            for hh in range(HQ):
                c0 = hh * DH
                q0 = qbuf[0:NGLOB, c0:c0 + DH]
                kh = kfull[:, c0:c0 + DH]
                s = lax.dot_general(
                    q0, kh, (((1,), (1,)), ((), ())),
                    preferred_element_type=jnp.float32) * SCALE
                m = jnp.max(s, axis=1)
                w = jnp.exp(s - m[:, None])
                denom = jnp.sum(w, axis=1)
                acc = lax.dot_general(
                    w.astype(jnp.bfloat16), vfull[:, c0:c0 + DH],
                    (((1,), (0,)), ((), ())),
                    preferred_element_type=jnp.float32)
                ctx[0:NGLOB, c0:c0 + DH] = acc / denom[:, None]

        out_ref[0] = jnp.dot(
            ctx[...].astype(jnp.bfloat16), wo_ref[...].astype(jnp.bfloat16),
            preferred_element_type=jnp.float32)

    return pl.pallas_call(
        body,
        out_shape=jax.ShapeDtypeStruct((1, S_PER, DM), jnp.float32),
        in_specs=[pl.BlockSpec(memory_space=pltpu.VMEM)] * 5,
        out_specs=pl.BlockSpec(memory_space=pltpu.VMEM),
        scratch_shapes=[
            pltpu.VMEM((S_GLOB, DM), jnp.bfloat16),
            pltpu.VMEM((S_GLOB, DM), jnp.bfloat16),
            pltpu.VMEM((S_PER, DM), jnp.bfloat16),
            pltpu.VMEM((S_PER, DM), jnp.float32),
            pltpu.SemaphoreType.DMA((2, N_DEV - 1)),
            pltpu.SemaphoreType.DMA((2, N_DEV - 1)),
        ],
        compiler_params=pltpu.CompilerParams(collective_id=0),
    )(x, Wq, k2, v2, Wo)


# baseline (device time: 104012 ns/iter reference)
import jax
import jax.numpy as jnp
from jax import lax
from jax.experimental import pallas as pl
from jax.experimental.pallas import tpu as pltpu

N_DEV = 4
S_PER = 2048
S_GLOB = N_DEV * S_PER
HQ = 8
DH = 128
DM = HQ * DH
QBLK = 256
WIN = QBLK + 256
HALO = 128
NGLOB = 32
SCALE = 0.08838834764831843
NEG = -1e9
N_SEND = 10


def kernel(x, Wq, K_ext, V_ext, Wo):
    xb = x.reshape(S_PER, DM).astype(jnp.bfloat16)
    kb = K_ext.reshape(S_PER, DM).astype(jnp.bfloat16)
    vb = V_ext.reshape(S_PER, DM).astype(jnp.bfloat16)
    wqb = Wq.astype(jnp.bfloat16)
    wob = Wo.astype(jnp.bfloat16)

    def body(x_ref, wq_ref, k_ref, v_ref, wo_ref, out_ref,
             qbuf, ctx, kext, vext, xg, kg, vg,
             po_own, ps_own, po_recv, ps_recv,
             send_sems, bcast_recv, halo_recv, part_recv):
        my = lax.axis_index("i")
        ring_r = lax.rem(my + 1, N_DEV)
        ring_l = lax.rem(my + N_DEV - 1, N_DEV)

        def start_send(src, dst, slot, rsem, tgt):
            pltpu.make_async_remote_copy(
                src_ref=src, dst_ref=dst,
                send_sem=send_sems.at[slot], recv_sem=rsem,
                device_id=(tgt,), device_id_type=pl.DeviceIdType.MESH,
            ).start()

        def wait_send(src, slot):
            pltpu.make_async_remote_copy(
                src_ref=src, dst_ref=src,
                send_sem=send_sems.at[slot], recv_sem=send_sems.at[slot],
                device_id=(my,), device_id_type=pl.DeviceIdType.MESH,
            ).wait_send()

        def wait_recv(dst, rsem):
            pltpu.make_async_remote_copy(
                src_ref=dst, dst_ref=dst,
                send_sem=rsem, recv_sem=rsem,
                device_id=(my,), device_id_type=pl.DeviceIdType.MESH,
            ).wait_recv()

        kext[pl.ds(HALO, S_PER), :] = k_ref[...]
        vext[pl.ds(HALO, S_PER), :] = v_ref[...]
        kext[pl.ds(0, HALO), :] = jnp.zeros((HALO, DM), jnp.bfloat16)
        vext[pl.ds(0, HALO), :] = jnp.zeros((HALO, DM), jnp.bfloat16)
        kext[pl.ds(HALO + S_PER, HALO), :] = jnp.zeros((HALO, DM), jnp.bfloat16)
        vext[pl.ds(HALO + S_PER, HALO), :] = jnp.zeros((HALO, DM), jnp.bfloat16)

        @pl.when(my == 0)
        def _():
            xg[...] = x_ref[0:NGLOB, :]
            kg[...] = k_ref[0:NGLOB, :]
            vg[...] = v_ref[0:NGLOB, :]

        barrier = pltpu.get_barrier_semaphore()
        for nbr in (ring_l, ring_r):
            pl.semaphore_signal(barrier, inc=1, device_id=(nbr,),
                                device_id_type=pl.DeviceIdType.MESH)
        pl.semaphore_wait(barrier, 2)

        @pl.when(my == 0)
        def _():
            start_send(xg, xg, 0, bcast_recv.at[0], 1)
            start_send(xg, xg, 1, bcast_recv.at[0], 3)
            start_send(kg, kg, 2, bcast_recv.at[1], 1)
            start_send(kg, kg, 3, bcast_recv.at[1], 3)
            start_send(vg, vg, 4, bcast_recv.at[2], 1)
            start_send(vg, vg, 5, bcast_recv.at[2], 3)

        @pl.when(my > 0)
        def _():
            start_send(kext.at[pl.ds(HALO, HALO), :],
                       kext.at[pl.ds(HALO + S_PER, HALO), :],
                       6, halo_recv.at[2], ring_l)
            start_send(vext.at[pl.ds(HALO, HALO), :],
                       vext.at[pl.ds(HALO + S_PER, HALO), :],
                       7, halo_recv.at[3], ring_l)

        @pl.when(my < N_DEV - 1)
        def _():
            start_send(kext.at[pl.ds(S_PER, HALO), :],
                       kext.at[pl.ds(0, HALO), :],
                       8, halo_recv.at[0], ring_r)
            start_send(vext.at[pl.ds(S_PER, HALO), :],
                       vext.at[pl.ds(0, HALO), :],
                       9, halo_recv.at[1], ring_r)

        @pl.when(my > 0)
        def _():
            wait_recv(xg, bcast_recv.at[0])
            wait_recv(kg, bcast_recv.at[1])
            wait_recv(vg, bcast_recv.at[2])

        @pl.when(my == 1)
        def _():
            start_send(xg, xg, 0, bcast_recv.at[0], 2)
            start_send(kg, kg, 1, bcast_recv.at[1], 2)
            start_send(vg, vg, 2, bcast_recv.at[2], 2)

        qg = jnp.dot(xg[...], wq_ref[...],
                     preferred_element_type=jnp.float32).astype(jnp.bfloat16)
        for hh in range(HQ):
            c0 = hh * DH
            s = lax.dot_general(
                qg[:, c0:c0 + DH], kext[pl.ds(HALO, S_PER), c0:c0 + DH],
                (((1,), (1,)), ((), ())),
                preferred_element_type=jnp.float32) * SCALE
            m = jnp.max(s, axis=1)
            w = jnp.exp(s - m[:, None])
            l = jnp.sum(w, axis=1)
            o = lax.dot_general(
                w.astype(jnp.bfloat16), vext[pl.ds(HALO, S_PER), c0:c0 + DH],
                (((1,), (0,)), ((), ())),
                preferred_element_type=jnp.float32)
            po_own[:, c0:c0 + DH] = o
            ps_own[hh, :] = m
            ps_own[HQ + hh, :] = l

        def merge_partial(slot):
            for hh in range(HQ):
                c0 = hh * DH
                m1 = ps_own[hh, :]
                l1 = ps_own[HQ + hh, :]
                m2 = ps_recv[slot, hh, :]
                l2 = ps_recv[slot, HQ + hh, :]
                mn = jnp.maximum(m1, m2)
                a1 = jnp.exp(m1 - mn)
                a2 = jnp.exp(m2 - mn)
                ps_own[hh, :] = mn
                ps_own[HQ + hh, :] = l1 * a1 + l2 * a2
                po_own[:, c0:c0 + DH] = (
                    po_own[:, c0:c0 + DH] * a1[:, None]
                    + po_recv[slot, :, c0:c0 + DH] * a2[:, None])

        @pl.when(my == 1)
        def _():
            start_send(po_own, po_recv.at[0], 3, part_recv.at[0, 0], 0)
            start_send(ps_own, ps_recv.at[0], 4, part_recv.at[0, 1], 0)

        @pl.when(my == 2)
        def _():
            start_send(po_own, po_recv.at[0], 3, part_recv.at[0, 0], 3)
            start_send(ps_own, ps_recv.at[0], 4, part_recv.at[0, 1], 3)

        @pl.when(my == 3)
        def _():
            wait_recv(po_recv.at[0], part_recv.at[0, 0])
            wait_recv(ps_recv.at[0], part_recv.at[0, 1])
            merge_partial(0)
            start_send(po_own, po_recv.at[1], 3, part_recv.at[1, 0], 0)
            start_send(ps_own, ps_recv.at[1], 4, part_recv.at[1, 1], 0)

        for r in range(0, S_PER, 1024):
            qbuf[pl.ds(r, 1024), :] = jnp.dot(
                x_ref[pl.ds(r, 1024), :], wq_ref[...],
                preferred_element_type=jnp.float32).astype(jnp.bfloat16)

        @pl.when(my > 0)
        def _():
            wait_recv(kext.at[pl.ds(0, HALO), :], halo_recv.at[0])
            wait_recv(vext.at[pl.ds(0, HALO), :], halo_recv.at[1])

        @pl.when(my < N_DEV - 1)
        def _():
            wait_recv(kext.at[pl.ds(HALO + S_PER, HALO), :], halo_recv.at[2])
            wait_recv(vext.at[pl.ds(HALO + S_PER, HALO), :], halo_recv.at[3])

        for hh in range(HQ):
            c0 = hh * DH
            kgh = kg[:, c0:c0 + DH]
            vgh = vg[:, c0:c0 + DH]

            def qblock(qb, _, c0=c0, kgh=kgh, vgh=vgh):
                sl = pl.multiple_of(qb * QBLK, QBLK)
                qh = qbuf[pl.ds(sl, QBLK), c0:c0 + DH]
                kw = kext[pl.ds(sl, WIN), c0:c0 + DH]
                vw = vext[pl.ds(sl, WIN), c0:c0 + DH]
                sb = lax.dot_general(
                    qh, kw, (((1,), (1,)), ((), ())),
                    preferred_element_type=jnp.float32) * SCALE
                sg = lax.dot_general(
                    qh, kgh, (((1,), (1,)), ((), ())),
                    preferred_element_type=jnp.float32) * SCALE
                off = my * S_PER + qb * QBLK
                wstart = off - HALO
                rows = off + lax.broadcasted_iota(jnp.int32, (QBLK, WIN), 0)
                cols = wstart + lax.broadcasted_iota(jnp.int32, (QBLK, WIN), 1)
                ok_b = ((jnp.abs(rows - cols) <= HALO) | (cols < NGLOB)) \
                    & (cols >= 0) & (cols < S_GLOB)
                sb = jnp.where(ok_b, sb, NEG)
                gcols = lax.broadcasted_iota(jnp.int32, (QBLK, NGLOB), 1)
                ok_g = (gcols < wstart) | (gcols >= wstart + WIN)
                sg = jnp.where(ok_g, sg, NEG)
                m = jnp.maximum(jnp.max(sb, axis=1), jnp.max(sg, axis=1))
                wb = jnp.exp(sb - m[:, None])
                wg = jnp.exp(sg - m[:, None])
                denom = jnp.sum(wb, axis=1) + jnp.sum(wg, axis=1)
                acc = lax.dot_general(
                    wb.astype(jnp.bfloat16), vw, (((1,), (0,)), ((), ())),
                    preferred_element_type=jnp.float32)
                acc += lax.dot_general(
                    wg.astype(jnp.bfloat16), vgh, (((1,), (0,)), ((), ())),
                    preferred_element_type=jnp.float32)
                ctx[pl.ds(sl, QBLK), c0:c0 + DH] = (
                    acc / denom[:, None]).astype(jnp.bfloat16)
                return 0

            lax.fori_loop(0, S_PER // QBLK, qblock, 0)

        @pl.when(my == 0)
        def _():
            wait_recv(po_recv.at[0], part_recv.at[0, 0])
            wait_recv(ps_recv.at[0], part_recv.at[0, 1])
            merge_partial(0)
            wait_recv(po_recv.at[1], part_recv.at[1, 0])
            wait_recv(ps_recv.at[1], part_recv.at[1, 1])
            merge_partial(1)
            for hh in range(HQ):
                c0 = hh * DH
                l = ps_own[HQ + hh, :]
                ctx[0:NGLOB, c0:c0 + DH] = (
                    po_own[:, c0:c0 + DH] / l[:, None]).astype(jnp.bfloat16)

        @pl.when(my == 0)
        def _():
            for src, slot in ((xg, 0), (xg, 1), (kg, 2), (kg, 3),
                              (vg, 4), (vg, 5)):
                wait_send(src, slot)

        @pl.when(my == 1)
        def _():
            for src, slot in ((xg, 0), (kg, 1), (vg, 2)):
                wait_send(src, slot)

        @pl.when((my == 1) | (my == 2) | (my == 3))
        def _():
            wait_send(po_own, 3)
            wait_send(ps_own, 4)
            wait_send(kext.at[pl.ds(HALO, HALO), :], 6)
            wait_send(vext.at[pl.ds(HALO, HALO), :], 7)

        @pl.when(my < N_DEV - 1)
        def _():
            wait_send(kext.at[pl.ds(S_PER, HALO), :], 8)
            wait_send(vext.at[pl.ds(S_PER, HALO), :], 9)

        for r in range(0, S_PER, 1024):
            out_ref[0, pl.ds(r, 1024), :] = jnp.dot(
                ctx[pl.ds(r, 1024), :], wo_ref[...],
                preferred_element_type=jnp.float32)

    return pl.pallas_call(
        body,
        out_shape=jax.ShapeDtypeStruct((1, S_PER, DM), jnp.float32),
        in_specs=[pl.BlockSpec(memory_space=pltpu.VMEM)] * 5,
        out_specs=pl.BlockSpec(memory_space=pltpu.VMEM),
        scratch_shapes=[
            pltpu.VMEM((S_PER, DM), jnp.bfloat16),
            pltpu.VMEM((S_PER, DM), jnp.bfloat16),
            pltpu.VMEM((S_PER + 2 * HALO, DM), jnp.bfloat16),
            pltpu.VMEM((S_PER + 2 * HALO, DM), jnp.bfloat16),
            pltpu.VMEM((NGLOB, DM), jnp.bfloat16),
            pltpu.VMEM((NGLOB, DM), jnp.bfloat16),
            pltpu.VMEM((NGLOB, DM), jnp.bfloat16),
            pltpu.VMEM((NGLOB, DM), jnp.float32),
            pltpu.VMEM((2 * HQ, NGLOB), jnp.float32),
            pltpu.VMEM((2, NGLOB, DM), jnp.float32),
            pltpu.VMEM((2, 2 * HQ, NGLOB), jnp.float32),
            pltpu.SemaphoreType.DMA((N_SEND,)),
            pltpu.SemaphoreType.DMA((3,)),
            pltpu.SemaphoreType.DMA((4,)),
            pltpu.SemaphoreType.DMA((2, 2)),
        ],
        compiler_params=pltpu.CompilerParams(collective_id=0),
    )(xb, wqb, kb, vb, wob)


# device time: 90326 ns/iter; 1.1515x vs baseline; 1.1515x over previous
import jax
import jax.numpy as jnp
from jax import lax
from jax.experimental import pallas as pl
from jax.experimental.pallas import tpu as pltpu

N_DEV = 4
S_PER = 2048
S_GLOB = N_DEV * S_PER
HQ = 8
DH = 128
DM = HQ * DH
QBLK = 256
WIN = QBLK + 256
HALO = 128
NGLOB = 32
SCALE = 0.08838834764831843
NEG = -1e9
N_SEND = 10


def kernel(x, Wq, K_ext, V_ext, Wo):
    xb = x.reshape(S_PER, DM).astype(jnp.bfloat16)
    kb = K_ext.reshape(S_PER, DM).astype(jnp.bfloat16)
    vb = V_ext.reshape(S_PER, DM).astype(jnp.bfloat16)
    wqb = Wq.astype(jnp.bfloat16)
    wob = Wo.astype(jnp.bfloat16)

    def body(x_ref, wq_ref, k_ref, v_ref, wo_ref, out_ref,
             qbuf, kext, vext, xg, kg, vg,
             po_own, ps_own, po_recv, ps_recv,
             send_sems, bcast_recv, halo_recv, part_recv):
        ctx = qbuf
        my = lax.axis_index("i")
        ring_r = lax.rem(my + 1, N_DEV)
        ring_l = lax.rem(my + N_DEV - 1, N_DEV)

        def start_send(src, dst, slot, rsem, tgt):
            pltpu.make_async_remote_copy(
                src_ref=src, dst_ref=dst,
                send_sem=send_sems.at[slot], recv_sem=rsem,
                device_id=(tgt,), device_id_type=pl.DeviceIdType.MESH,
            ).start()

        def wait_send(src, slot):
            pltpu.make_async_remote_copy(
                src_ref=src, dst_ref=src,
                send_sem=send_sems.at[slot], recv_sem=send_sems.at[slot],
                device_id=(my,), device_id_type=pl.DeviceIdType.MESH,
            ).wait_send()

        def wait_recv(dst, rsem):
            pltpu.make_async_remote_copy(
                src_ref=dst, dst_ref=dst,
                send_sem=rsem, recv_sem=rsem,
                device_id=(my,), device_id_type=pl.DeviceIdType.MESH,
            ).wait_recv()

        kext[pl.ds(HALO, S_PER), :] = k_ref[...]
        vext[pl.ds(HALO, S_PER), :] = v_ref[...]
        kext[pl.ds(0, HALO), :] = jnp.zeros((HALO, DM), jnp.bfloat16)
        vext[pl.ds(0, HALO), :] = jnp.zeros((HALO, DM), jnp.bfloat16)
        kext[pl.ds(HALO + S_PER, HALO), :] = jnp.zeros((HALO, DM), jnp.bfloat16)
        vext[pl.ds(HALO + S_PER, HALO), :] = jnp.zeros((HALO, DM), jnp.bfloat16)

        @pl.when(my == 0)
        def _():
            xg[...] = x_ref[0:NGLOB, :]
            kg[...] = k_ref[0:NGLOB, :]
            vg[...] = v_ref[0:NGLOB, :]

        barrier = pltpu.get_barrier_semaphore()
        for nbr in (ring_l, ring_r):
            pl.semaphore_signal(barrier, inc=1, device_id=(nbr,),
                                device_id_type=pl.DeviceIdType.MESH)
        pl.semaphore_wait(barrier, 2)

        @pl.when(my == 0)
        def _():
            start_send(xg, xg, 0, bcast_recv.at[0], 1)
            start_send(xg, xg, 1, bcast_recv.at[0], 3)
            start_send(kg, kg, 2, bcast_recv.at[1], 1)
            start_send(kg, kg, 3, bcast_recv.at[1], 3)
            start_send(vg, vg, 4, bcast_recv.at[2], 1)
            start_send(vg, vg, 5, bcast_recv.at[2], 3)

        @pl.when(my > 0)
        def _():
            start_send(kext.at[pl.ds(HALO, HALO), :],
                       kext.at[pl.ds(HALO + S_PER, HALO), :],
                       6, halo_recv.at[2], ring_l)
            start_send(vext.at[pl.ds(HALO, HALO), :],
                       vext.at[pl.ds(HALO + S_PER, HALO), :],
                       7, halo_recv.at[3], ring_l)

        @pl.when(my < N_DEV - 1)
        def _():
            start_send(kext.at[pl.ds(S_PER, HALO), :],
                       kext.at[pl.ds(0, HALO), :],
                       8, halo_recv.at[0], ring_r)
            start_send(vext.at[pl.ds(S_PER, HALO), :],
                       vext.at[pl.ds(0, HALO), :],
                       9, halo_recv.at[1], ring_r)

        @pl.when(my > 0)
        def _():
            wait_recv(xg, bcast_recv.at[0])
            wait_recv(kg, bcast_recv.at[1])
            wait_recv(vg, bcast_recv.at[2])

        @pl.when(my == 1)
        def _():
            start_send(xg, xg, 0, bcast_recv.at[0], 2)
            start_send(kg, kg, 1, bcast_recv.at[1], 2)
            start_send(vg, vg, 2, bcast_recv.at[2], 2)

        qg = jnp.dot(xg[...], wq_ref[...],
                     preferred_element_type=jnp.float32).astype(jnp.bfloat16)
        for hh in range(HQ):
            c0 = hh * DH
            s = lax.dot_general(
                qg[:, c0:c0 + DH], kext[pl.ds(HALO, S_PER), c0:c0 + DH],
                (((1,), (1,)), ((), ())),
                preferred_element_type=jnp.float32) * SCALE
            m = jnp.max(s, axis=1)
            w = jnp.exp(s - m[:, None])
            l = jnp.sum(w, axis=1)
            o = lax.dot_general(
                w.astype(jnp.bfloat16), vext[pl.ds(HALO, S_PER), c0:c0 + DH],
                (((1,), (0,)), ((), ())),
                preferred_element_type=jnp.float32)
            po_own[:, c0:c0 + DH] = o
            ps_own[hh, :] = m
            ps_own[HQ + hh, :] = l

        def merge_partial(slot):
            for hh in range(HQ):
                c0 = hh * DH
                m1 = ps_own[hh, :]
                l1 = ps_own[HQ + hh, :]
                m2 = ps_recv[slot, hh, :]
                l2 = ps_recv[slot, HQ + hh, :]
                mn = jnp.maximum(m1, m2)
                a1 = jnp.exp(m1 - mn)
                a2 = jnp.exp(m2 - mn)
                ps_own[hh, :] = mn
                ps_own[HQ + hh, :] = l1 * a1 + l2 * a2
                po_own[:, c0:c0 + DH] = (
                    po_own[:, c0:c0 + DH] * a1[:, None]
                    + po_recv[slot, :, c0:c0 + DH] * a2[:, None])

        @pl.when(my == 1)
        def _():
            start_send(po_own, po_recv.at[0], 3, part_recv.at[0, 0], 0)
            start_send(ps_own, ps_recv.at[0], 4, part_recv.at[0, 1], 0)

        @pl.when(my == 2)
        def _():
            start_send(po_own, po_recv.at[0], 3, part_recv.at[0, 0], 3)
            start_send(ps_own, ps_recv.at[0], 4, part_recv.at[0, 1], 3)

        @pl.when(my == 3)
        def _():
            wait_recv(po_recv.at[0], part_recv.at[0, 0])
            wait_recv(ps_recv.at[0], part_recv.at[0, 1])
            merge_partial(0)
            start_send(po_own, po_recv.at[1], 3, part_recv.at[1, 0], 0)
            start_send(ps_own, ps_recv.at[1], 4, part_recv.at[1, 1], 0)

        for r in range(0, S_PER, 1024):
            qbuf[pl.ds(r, 1024), :] = jnp.dot(
                x_ref[pl.ds(r, 1024), :], wq_ref[...],
                preferred_element_type=jnp.float32).astype(jnp.bfloat16)

        @pl.when(my > 0)
        def _():
            wait_recv(kext.at[pl.ds(0, HALO), :], halo_recv.at[0])
            wait_recv(vext.at[pl.ds(0, HALO), :], halo_recv.at[1])

        @pl.when(my < N_DEV - 1)
        def _():
            wait_recv(kext.at[pl.ds(HALO + S_PER, HALO), :], halo_recv.at[2])
            wait_recv(vext.at[pl.ds(HALO + S_PER, HALO), :], halo_recv.at[3])

        ii = lax.broadcasted_iota(jnp.int32, (QBLK, WIN), 0)
        jj = lax.broadcasted_iota(jnp.int32, (QBLK, WIN), 1)
        band_const = jnp.abs(HALO + ii - jj) <= HALO
        gcols = lax.broadcasted_iota(jnp.int32, (QBLK, NGLOB), 1)

        def attn_block(sl, ok_b, ok_g):
            for hh in range(HQ):
                c0 = hh * DH
                qh = qbuf[pl.ds(sl, QBLK), c0:c0 + DH]
                kw = kext[pl.ds(sl, WIN), c0:c0 + DH]
                vw = vext[pl.ds(sl, WIN), c0:c0 + DH]
                sb = lax.dot_general(
                    qh, kw, (((1,), (1,)), ((), ())),
                    preferred_element_type=jnp.float32) * SCALE
                sg = lax.dot_general(
                    qh, kg[:, c0:c0 + DH], (((1,), (1,)), ((), ())),
                    preferred_element_type=jnp.float32) * SCALE
                sb = jnp.where(ok_b, sb, NEG)
                if ok_g is not None:
                    sg = jnp.where(ok_g, sg, NEG)
                m = jnp.maximum(jnp.max(sb, axis=1), jnp.max(sg, axis=1))
                wb = jnp.exp(sb - m[:, None])
                wg = jnp.exp(sg - m[:, None])
                denom = jnp.sum(wb, axis=1) + jnp.sum(wg, axis=1)
                acc = lax.dot_general(
                    wb.astype(jnp.bfloat16), vw, (((1,), (0,)), ((), ())),
                    preferred_element_type=jnp.float32)
                acc += lax.dot_general(
                    wg.astype(jnp.bfloat16), vg[:, c0:c0 + DH],
                    (((1,), (0,)), ((), ())),
                    preferred_element_type=jnp.float32)
                ctx[pl.ds(sl, QBLK), c0:c0 + DH] = (
                    acc / denom[:, None]).astype(jnp.bfloat16)

        def interior(qb, _):
            attn_block(pl.multiple_of(qb * QBLK, QBLK), band_const, None)
            return 0

        lax.fori_loop(1, S_PER // QBLK - 1, interior, 0)

        for qb in (0, S_PER // QBLK - 1):
            sl = qb * QBLK
            wstart = my * S_PER + sl - HALO
            cols = wstart + jj
            ok_b = ((band_const | (cols < NGLOB))
                    & (cols >= 0) & (cols < S_GLOB))
            ok_g = (gcols < wstart) | (gcols >= wstart + WIN)
            attn_block(sl, ok_b, ok_g)

        @pl.when(my == 0)
        def _():
            wait_recv(po_recv.at[0], part_recv.at[0, 0])
            wait_recv(ps_recv.at[0], part_recv.at[0, 1])
            merge_partial(0)
            wait_recv(po_recv.at[1], part_recv.at[1, 0])
            wait_recv(ps_recv.at[1], part_recv.at[1, 1])
            merge_partial(1)
            for hh in range(HQ):
                c0 = hh * DH
                l = ps_own[HQ + hh, :]
                ctx[0:NGLOB, c0:c0 + DH] = (
                    po_own[:, c0:c0 + DH] / l[:, None]).astype(jnp.bfloat16)

        @pl.when(my == 0)
        def _():
            for src, slot in ((xg, 0), (xg, 1), (kg, 2), (kg, 3),
                              (vg, 4), (vg, 5)):
                wait_send(src, slot)

        @pl.when(my == 1)
        def _():
            for src, slot in ((xg, 0), (kg, 1), (vg, 2)):
                wait_send(src, slot)

        @pl.when((my == 1) | (my == 2) | (my == 3))
        def _():
            wait_send(po_own, 3)
            wait_send(ps_own, 4)
            wait_send(kext.at[pl.ds(HALO, HALO), :], 6)
            wait_send(vext.at[pl.ds(HALO, HALO), :], 7)

        @pl.when(my < N_DEV - 1)
        def _():
            wait_send(kext.at[pl.ds(S_PER, HALO), :], 8)
            wait_send(vext.at[pl.ds(S_PER, HALO), :], 9)

        for r in range(0, S_PER, 1024):
            out_ref[0, pl.ds(r, 1024), :] = jnp.dot(
                ctx[pl.ds(r, 1024), :], wo_ref[...],
                preferred_element_type=jnp.float32)

    return pl.pallas_call(
        body,
        out_shape=jax.ShapeDtypeStruct((1, S_PER, DM), jnp.float32),
        in_specs=[pl.BlockSpec(memory_space=pltpu.VMEM)] * 5,
        out_specs=pl.BlockSpec(memory_space=pltpu.VMEM),
        scratch_shapes=[
            pltpu.VMEM((S_PER, DM), jnp.bfloat16),
            pltpu.VMEM((S_PER + 2 * HALO, DM), jnp.bfloat16),
            pltpu.VMEM((S_PER + 2 * HALO, DM), jnp.bfloat16),
            pltpu.VMEM((NGLOB, DM), jnp.bfloat16),
            pltpu.VMEM((NGLOB, DM), jnp.bfloat16),
            pltpu.VMEM((NGLOB, DM), jnp.bfloat16),
            pltpu.VMEM((NGLOB, DM), jnp.float32),
            pltpu.VMEM((2 * HQ, NGLOB), jnp.float32),
            pltpu.VMEM((2, NGLOB, DM), jnp.float32),
            pltpu.VMEM((2, 2 * HQ, NGLOB), jnp.float32),
            pltpu.SemaphoreType.DMA((N_SEND,)),
            pltpu.SemaphoreType.DMA((3,)),
            pltpu.SemaphoreType.DMA((4,)),
            pltpu.SemaphoreType.DMA((2, 2)),
        ],
        compiler_params=pltpu.CompilerParams(
            collective_id=0, vmem_limit_bytes=44 * 1024 * 1024),
    )(xb, wqb, kb, vb, wob)


# device time: 90072 ns/iter; 1.1548x vs baseline; 1.0028x over previous
import jax
import jax.numpy as jnp
from jax import lax
from jax.experimental import pallas as pl
from jax.experimental.pallas import tpu as pltpu

N_DEV = 4
S_PER = 2048
S_GLOB = N_DEV * S_PER
HQ = 8
DH = 128
DM = HQ * DH
QBLK = 256
WIN = QBLK + 256
HALO = 128
NGLOB = 32
SCALE = 0.08838834764831843
NEG = -1e9
N_SEND = 10


def kernel(x, Wq, K_ext, V_ext, Wo):
    xb = x.reshape(S_PER, DM).astype(jnp.bfloat16)
    kb = K_ext.reshape(S_PER, DM).astype(jnp.bfloat16)
    vb = V_ext.reshape(S_PER, DM).astype(jnp.bfloat16)
    wqb = Wq.astype(jnp.bfloat16)
    wob = Wo.astype(jnp.bfloat16)

    def body(x_ref, wq_ref, k_ref, v_ref, wo_ref, out_ref,
             qbuf, kext, vext, xg, kg, vg,
             po_own, ps_own, po_recv, ps_recv,
             send_sems, bcast_recv, halo_recv, part_recv):
        ctx = qbuf
        my = lax.axis_index("i")
        ring_r = lax.rem(my + 1, N_DEV)
        ring_l = lax.rem(my + N_DEV - 1, N_DEV)

        def start_send(src, dst, slot, rsem, tgt):
            pltpu.make_async_remote_copy(
                src_ref=src, dst_ref=dst,
                send_sem=send_sems.at[slot], recv_sem=rsem,
                device_id=(tgt,), device_id_type=pl.DeviceIdType.MESH,
            ).start()

        def wait_send(src, slot):
            pltpu.make_async_remote_copy(
                src_ref=src, dst_ref=src,
                send_sem=send_sems.at[slot], recv_sem=send_sems.at[slot],
                device_id=(my,), device_id_type=pl.DeviceIdType.MESH,
            ).wait_send()

        def wait_recv(dst, rsem):
            pltpu.make_async_remote_copy(
                src_ref=dst, dst_ref=dst,
                send_sem=rsem, recv_sem=rsem,
                device_id=(my,), device_id_type=pl.DeviceIdType.MESH,
            ).wait_recv()

        kext[pl.ds(HALO, S_PER), :] = k_ref[...]
        vext[pl.ds(HALO, S_PER), :] = v_ref[...]
        kext[pl.ds(0, HALO), :] = jnp.zeros((HALO, DM), jnp.bfloat16)
        vext[pl.ds(0, HALO), :] = jnp.zeros((HALO, DM), jnp.bfloat16)
        kext[pl.ds(HALO + S_PER, HALO), :] = jnp.zeros((HALO, DM), jnp.bfloat16)
        vext[pl.ds(HALO + S_PER, HALO), :] = jnp.zeros((HALO, DM), jnp.bfloat16)

        @pl.when(my == 0)
        def _():
            xg[...] = x_ref[0:NGLOB, :]
            kg[...] = k_ref[0:NGLOB, :]
            vg[...] = v_ref[0:NGLOB, :]

        barrier = pltpu.get_barrier_semaphore()
        for nbr in (ring_l, ring_r):
            pl.semaphore_signal(barrier, inc=1, device_id=(nbr,),
                                device_id_type=pl.DeviceIdType.MESH)
        pl.semaphore_wait(barrier, 2)

        @pl.when(my == 0)
        def _():
            start_send(xg, xg, 0, bcast_recv.at[0], 1)
            start_send(xg, xg, 1, bcast_recv.at[0], 3)
            start_send(kg, kg, 2, bcast_recv.at[1], 1)
            start_send(kg, kg, 3, bcast_recv.at[1], 3)
            start_send(vg, vg, 4, bcast_recv.at[2], 1)
            start_send(vg, vg, 5, bcast_recv.at[2], 3)

        @pl.when(my > 0)
        def _():
            start_send(kext.at[pl.ds(HALO, HALO), :],
                       kext.at[pl.ds(HALO + S_PER, HALO), :],
                       6, halo_recv.at[2], ring_l)
            start_send(vext.at[pl.ds(HALO, HALO), :],
                       vext.at[pl.ds(HALO + S_PER, HALO), :],
                       7, halo_recv.at[3], ring_l)

        @pl.when(my < N_DEV - 1)
        def _():
            start_send(kext.at[pl.ds(S_PER, HALO), :],
                       kext.at[pl.ds(0, HALO), :],
                       8, halo_recv.at[0], ring_r)
            start_send(vext.at[pl.ds(S_PER, HALO), :],
                       vext.at[pl.ds(0, HALO), :],
                       9, halo_recv.at[1], ring_r)

        @pl.when(my > 0)
        def _():
            wait_recv(xg, bcast_recv.at[0])
            wait_recv(kg, bcast_recv.at[1])
            wait_recv(vg, bcast_recv.at[2])

        @pl.when(my == 1)
        def _():
            start_send(xg, xg, 0, bcast_recv.at[0], 2)
            start_send(kg, kg, 1, bcast_recv.at[1], 2)
            start_send(vg, vg, 2, bcast_recv.at[2], 2)

        qg = jnp.dot(xg[...], wq_ref[...],
                     preferred_element_type=jnp.float32).astype(jnp.bfloat16)
        for hh in range(HQ):
            c0 = hh * DH
            s = lax.dot_general(
                qg[:, c0:c0 + DH], kext[pl.ds(HALO, S_PER), c0:c0 + DH],
                (((1,), (1,)), ((), ())),
                preferred_element_type=jnp.float32) * SCALE
            m = jnp.max(s, axis=1)
            w = jnp.exp(s - m[:, None])
            l = jnp.sum(w, axis=1)
            o = lax.dot_general(
                w.astype(jnp.bfloat16), vext[pl.ds(HALO, S_PER), c0:c0 + DH],
                (((1,), (0,)), ((), ())),
                preferred_element_type=jnp.float32)
            po_own[:, c0:c0 + DH] = o
            ps_own[hh, :] = m
            ps_own[HQ + hh, :] = l

        def merge_partial(slot):
            for hh in range(HQ):
                c0 = hh * DH
                m1 = ps_own[hh, :]
                l1 = ps_own[HQ + hh, :]
                m2 = ps_recv[slot, hh, :]
                l2 = ps_recv[slot, HQ + hh, :]
                mn = jnp.maximum(m1, m2)
                a1 = jnp.exp(m1 - mn)
                a2 = jnp.exp(m2 - mn)
                ps_own[hh, :] = mn
                ps_own[HQ + hh, :] = l1 * a1 + l2 * a2
                po_own[:, c0:c0 + DH] = (
                    po_own[:, c0:c0 + DH] * a1[:, None]
                    + po_recv[slot, :, c0:c0 + DH] * a2[:, None])

        @pl.when(my == 1)
        def _():
            start_send(po_own, po_recv.at[0], 3, part_recv.at[0, 0], 0)
            start_send(ps_own, ps_recv.at[0], 4, part_recv.at[0, 1], 0)

        @pl.when(my == 2)
        def _():
            start_send(po_own, po_recv.at[0], 3, part_recv.at[0, 0], 3)
            start_send(ps_own, ps_recv.at[0], 4, part_recv.at[0, 1], 3)

        @pl.when(my == 3)
        def _():
            wait_recv(po_recv.at[0], part_recv.at[0, 0])
            wait_recv(ps_recv.at[0], part_recv.at[0, 1])
            merge_partial(0)
            start_send(po_own, po_recv.at[1], 3, part_recv.at[1, 0], 0)
            start_send(ps_own, ps_recv.at[1], 4, part_recv.at[1, 1], 0)

        for r in range(0, S_PER, 1024):
            qbuf[pl.ds(r, 1024), :] = jnp.dot(
                x_ref[pl.ds(r, 1024), :], wq_ref[...],
                preferred_element_type=jnp.float32).astype(jnp.bfloat16)

        ii = lax.broadcasted_iota(jnp.int32, (QBLK, WIN), 0)
        jj = lax.broadcasted_iota(jnp.int32, (QBLK, WIN), 1)
        band_const = jnp.abs(HALO + ii - jj) <= HALO
        gcols = lax.broadcasted_iota(jnp.int32, (QBLK, NGLOB), 1)

        def attn_block(sl, ok_b, ok_g):
            for hh in range(HQ):
                c0 = hh * DH
                qh = qbuf[pl.ds(sl, QBLK), c0:c0 + DH]
                kw = kext[pl.ds(sl, WIN), c0:c0 + DH]
                vw = vext[pl.ds(sl, WIN), c0:c0 + DH]
                sb = lax.dot_general(
                    qh, kw, (((1,), (1,)), ((), ())),
                    preferred_element_type=jnp.float32) * SCALE
                sg = lax.dot_general(
                    qh, kg[:, c0:c0 + DH], (((1,), (1,)), ((), ())),
                    preferred_element_type=jnp.float32) * SCALE
                sb = jnp.where(ok_b, sb, NEG)
                if ok_g is not None:
                    sg = jnp.where(ok_g, sg, NEG)
                m = jnp.maximum(jnp.max(sb, axis=1), jnp.max(sg, axis=1))
                wb = jnp.exp(sb - m[:, None])
                wg = jnp.exp(sg - m[:, None])
                denom = jnp.sum(wb, axis=1) + jnp.sum(wg, axis=1)
                acc = lax.dot_general(
                    wb.astype(jnp.bfloat16), vw, (((1,), (0,)), ((), ())),
                    preferred_element_type=jnp.float32)
                acc += lax.dot_general(
                    wg.astype(jnp.bfloat16), vg[:, c0:c0 + DH],
                    (((1,), (0,)), ((), ())),
                    preferred_element_type=jnp.float32)
                ctx[pl.ds(sl, QBLK), c0:c0 + DH] = (
                    acc / denom[:, None]).astype(jnp.bfloat16)

        def interior(qb, _):
            attn_block(pl.multiple_of(qb * QBLK, QBLK), band_const, None)
            return 0

        lax.fori_loop(1, S_PER // QBLK - 1, interior, 0)

        @pl.when(my > 0)
        def _():
            wait_recv(kext.at[pl.ds(0, HALO), :], halo_recv.at[0])
            wait_recv(vext.at[pl.ds(0, HALO), :], halo_recv.at[1])

        @pl.when(my < N_DEV - 1)
        def _():
            wait_recv(kext.at[pl.ds(HALO + S_PER, HALO), :], halo_recv.at[2])
            wait_recv(vext.at[pl.ds(HALO + S_PER, HALO), :], halo_recv.at[3])

        for qb in (0, S_PER // QBLK - 1):
            sl = qb * QBLK
            wstart = my * S_PER + sl - HALO
            cols = wstart + jj
            ok_b = ((band_const | (cols < NGLOB))
                    & (cols >= 0) & (cols < S_GLOB))
            ok_g = (gcols < wstart) | (gcols >= wstart + WIN)
            attn_block(sl, ok_b, ok_g)

        @pl.when(my == 0)
        def _():
            wait_recv(po_recv.at[0], part_recv.at[0, 0])
            wait_recv(ps_recv.at[0], part_recv.at[0, 1])
            merge_partial(0)
            wait_recv(po_recv.at[1], part_recv.at[1, 0])
            wait_recv(ps_recv.at[1], part_recv.at[1, 1])
            merge_partial(1)
            for hh in range(HQ):
                c0 = hh * DH
                l = ps_own[HQ + hh, :]
                ctx[0:NGLOB, c0:c0 + DH] = (
                    po_own[:, c0:c0 + DH] / l[:, None]).astype(jnp.bfloat16)

        @pl.when(my == 0)
        def _():
            for src, slot in ((xg, 0), (xg, 1), (kg, 2), (kg, 3),
                              (vg, 4), (vg, 5)):
                wait_send(src, slot)

        @pl.when(my == 1)
        def _():
            for src, slot in ((xg, 0), (kg, 1), (vg, 2)):
                wait_send(src, slot)

        @pl.when((my == 1) | (my == 2) | (my == 3))
        def _():
            wait_send(po_own, 3)
            wait_send(ps_own, 4)
            wait_send(kext.at[pl.ds(HALO, HALO), :], 6)
            wait_send(vext.at[pl.ds(HALO, HALO), :], 7)

        @pl.when(my < N_DEV - 1)
        def _():
            wait_send(kext.at[pl.ds(S_PER, HALO), :], 8)
            wait_send(vext.at[pl.ds(S_PER, HALO), :], 9)

        for r in range(0, S_PER, 1024):
            out_ref[0, pl.ds(r, 1024), :] = jnp.dot(
                ctx[pl.ds(r, 1024), :], wo_ref[...],
                preferred_element_type=jnp.float32)

    return pl.pallas_call(
        body,
        out_shape=jax.ShapeDtypeStruct((1, S_PER, DM), jnp.float32),
        in_specs=[pl.BlockSpec(memory_space=pltpu.VMEM)] * 5,
        out_specs=pl.BlockSpec(memory_space=pltpu.VMEM),
        scratch_shapes=[
            pltpu.VMEM((S_PER, DM), jnp.bfloat16),
            pltpu.VMEM((S_PER + 2 * HALO, DM), jnp.bfloat16),
            pltpu.VMEM((S_PER + 2 * HALO, DM), jnp.bfloat16),
            pltpu.VMEM((NGLOB, DM), jnp.bfloat16),
            pltpu.VMEM((NGLOB, DM), jnp.bfloat16),
            pltpu.VMEM((NGLOB, DM), jnp.bfloat16),
            pltpu.VMEM((NGLOB, DM), jnp.float32),
            pltpu.VMEM((2 * HQ, NGLOB), jnp.float32),
            pltpu.VMEM((2, NGLOB, DM), jnp.float32),
            pltpu.VMEM((2, 2 * HQ, NGLOB), jnp.float32),
            pltpu.SemaphoreType.DMA((N_SEND,)),
            pltpu.SemaphoreType.DMA((3,)),
            pltpu.SemaphoreType.DMA((4,)),
            pltpu.SemaphoreType.DMA((2, 2)),
        ],
        compiler_params=pltpu.CompilerParams(
            collective_id=0, vmem_limit_bytes=44 * 1024 * 1024),
    )(xb, wqb, kb, vb, wob)
